# baseline (speedup 1.0000x reference)
import jax
import jax.numpy as jnp
from jax.experimental import pallas as pl
from jax.experimental.pallas import tpu as pltpu

_SKIP = 5
_H, _W = 480, 640
_PY, _PX = _H // _SKIP, _W // _SKIP
_CSTRIDE = _SKIP * 4
_QY, _QX = _H // _CSTRIDE, _W // _CSTRIDE
_Q = _QY * _QX
_NK = 2 * _QY - 1
_RPB = 8
_NSTEPS = _PY // _RPB
_HROWS = _RPB * _SKIP

_INLIER_T = 0.5
_ENC2 = 128.0


def _mm(a, b):
    return jax.lax.dot_general(
        a, b, (((1,), (0,)), ((), ())),
        precision=jax.lax.Precision.HIGHEST,
        preferred_element_type=jnp.float32,
    )


def _vote_body(lab_ref, vert_ref, meta_ref, ext_ref, out_ref,
               dnx_ref, dny_ref, acc_ref, sr_ref, sc_ref,
               labc_ref, uxc_ref, uyc_ref, vzc_ref):
    i = pl.program_id(0)

    @pl.when(i == 0)
    def _init():
        shape = (4, _NK, _QX, _PX)
        b_i = jax.lax.broadcasted_iota(jnp.int32, shape, 0)
        k_i = jax.lax.broadcasted_iota(jnp.int32, shape, 1)
        qx_i = jax.lax.broadcasted_iota(jnp.int32, shape, 2)
        px_i = jax.lax.broadcasted_iota(jnp.int32, shape, 3)
        dyv = (_CSTRIDE * k_i - _CSTRIDE * (_QY - 1) - _SKIP * b_i).astype(
            jnp.float32
        )
        dxv = (_CSTRIDE * qx_i - _SKIP * px_i).astype(jnp.float32)
        dnorm = jnp.sqrt(dxv * dxv + dyv * dyv) + 1e-8
        dnx_ref[...] = dxv / dnorm
        dny_ref[...] = dyv / dnorm
        acc_ref[...] = jnp.zeros_like(acc_ref)

        r_i = jax.lax.broadcasted_iota(jnp.int32, (_RPB, _HROWS), 0)
        x_i = jax.lax.broadcasted_iota(jnp.int32, (_RPB, _HROWS), 1)
        sr_ref[...] = (x_i == _SKIP * r_i).astype(jnp.float32)
        w_i = jax.lax.broadcasted_iota(jnp.int32, (_W, _PX), 0)
        j_i = jax.lax.broadcasted_iota(jnp.int32, (_W, _PX), 1)
        sc_ref[...] = (w_i == _SKIP * j_i).astype(jnp.float32)

    srm = sr_ref[...]
    scm = sc_ref[...]
    lab8 = _mm(_mm(srm, lab_ref[...].astype(jnp.float32)), scm)
    vx1 = _mm(_mm(srm, vert_ref[3]), scm)
    vy1 = _mm(_mm(srm, vert_ref[4]), scm)
    vz1 = _mm(_mm(srm, vert_ref[5]), scm)
    vx2 = _mm(_mm(srm, vert_ref[6]), scm)
    vy2 = _mm(_mm(srm, vert_ref[7]), scm)
    vz2 = _mm(_mm(srm, vert_ref[8]), scm)

    vn1 = jnp.sqrt(vx1 * vx1 + vy1 * vy1) + 1e-8
    vn2 = jnp.sqrt(vx2 * vx2 + vy2 * vy2) + 1e-8
    ux1 = vx1 / vn1
    uy1 = vy1 / vn1
    ux2 = vx2 / vn2
    uy2 = vy2 / vn2
    is1 = lab8 == 1.0
    is2 = lab8 == 2.0
    upx8 = jnp.where(is1, ux1, jnp.where(is2, ux2, 0.0))
    upy8 = jnp.where(is1, uy1, jnp.where(is2, uy2, 0.0))
    encv8 = jnp.where(is1, 1.0, jnp.where(is2, _ENC2, 0.0))

    rows = pl.ds(_RPB * i, _RPB)
    labc_ref[rows, :] = lab8
    uxc_ref[0, rows, :] = ux1
    uxc_ref[1, rows, :] = ux2
    uyc_ref[0, rows, :] = uy1
    uyc_ref[1, rows, :] = uy2
    vzc_ref[0, rows, :] = vz1
    vzc_ref[1, rows, :] = vz2

    for r in range(_RPB):
        b = r % 4
        a = 2 * i + r // 4
        dnx = dnx_ref[b, pl.ds(_QY - 1 - a, _QY)]
        dny = dny_ref[b, pl.ds(_QY - 1 - a, _QY)]
        upx = upx8[r : r + 1, :].reshape(1, 1, _PX)
        upy = upy8[r : r + 1, :].reshape(1, 1, _PX)
        encv = encv8[r : r + 1, :].reshape(1, 1, _PX)
        dots = dnx * upx + dny * upy
        acc_ref[...] += jnp.where(dots > _INLIER_T, encv, 0.0)

    @pl.when(i == _NSTEPS - 1)
    def _finish():
        fx = jnp.abs(meta_ref[0:1, 0:1]) + 1.0
        fy = jnp.abs(meta_ref[0:1, 1:2]) + 1.0
        qi = jax.lax.broadcasted_iota(jnp.int32, (_Q, 1), 0)
        lane = jax.lax.broadcasted_iota(jnp.int32, (1, 128), 1)
        z0 = jnp.zeros((1, 128), jnp.float32)

        accf = acc_ref[...].reshape(_Q, _PX)
        v2l = jnp.floor(accf * (1.0 / _ENC2))
        v1l = accf - _ENC2 * v2l

        labf = labc_ref[...]
        pxf = (_SKIP * jax.lax.broadcasted_iota(
            jnp.int32, (_PY, _PX), 1)).astype(jnp.float32)
        pyf = (_SKIP * jax.lax.broadcasted_iota(
            jnp.int32, (_PY, _PX), 0)).astype(jnp.float32)

        for c, vl in ((1, v1l), (2, v2l)):
            votes = jnp.sum(vl, axis=1, keepdims=True)
            m = jnp.max(votes, axis=0, keepdims=True)
            best = jnp.min(
                jnp.where(votes == m, qi, 2**30), axis=0, keepdims=True
            )

            uxf = uxc_ref[c - 1]
            uyf = uyc_ref[c - 1]
            vzf = vzc_ref[c - 1]
            maskf = labf == jnp.float32(c)
            bqx = (_CSTRIDE * (best % _QX)).astype(jnp.float32)
            bqy = (_CSTRIDE * (best // _QX)).astype(jnp.float32)
            dxb = bqx - pxf
            dyb = bqy - pyf
            dnb = jnp.sqrt(dxb * dxb + dyb * dyb) + 1e-8
            dotb = (dxb / dnb) * uxf + (dyb / dnb) * uyf
            hitb = (dotb > _INLIER_T) & maskf
            bvz = jnp.sum(
                jnp.sum(jnp.where(hitb, vzf, 0.0), axis=1, keepdims=True),
                axis=0, keepdims=True,
            )
            cnt = jnp.sum(
                jnp.sum(jnp.where(maskf, 1.0, 0.0), axis=1, keepdims=True),
                axis=0, keepdims=True,
            )

            mean_vz = bvz / (m + 1e-8)
            depth = jnp.exp(jnp.clip(mean_vz, -10.0, 10.0))
            e0 = ext_ref[c : c + 1, 0:1]
            e1 = ext_ref[c : c + 1, 1:2]
            e2 = ext_ref[c : c + 1, 2:3]
            diam = jnp.sqrt(e0 * e0 + e1 * e1 + e2 * e2) + 1e-8
            bw = diam * fx / (depth + 1e-8)
            bh = diam * fy / (depth + 1e-8)
            accept = (
                (m > 0.3 * (cnt + 1e-8)) & (cnt > 5.0)
            ).astype(jnp.float32)
            score = m / (cnt + 1e-8)

            roi = (
                jnp.where(lane == 1, jnp.float32(c), z0)
                + jnp.where(lane == 2, bqx - bw / 2.0, z0)
                + jnp.where(lane == 3, bqy - bh / 2.0, z0)
                + jnp.where(lane == 4, bqx + bw / 2.0, z0)
                + jnp.where(lane == 5, bqy + bh / 2.0, z0)
                + jnp.where(lane == 6, score * accept, z0)
            )
            out_ref[c - 1 : c, :] = roi


@jax.jit
def kernel(bottom_label, bottom_vertex, bottom_meta_data, extents):
    lab = bottom_label[0].astype(jnp.int32)
    vert = bottom_vertex[0].astype(jnp.float32)
    meta = bottom_meta_data.astype(jnp.float32)
    ext = extents.astype(jnp.float32)

    out = pl.pallas_call(
        _vote_body,
        grid=(_NSTEPS,),
        in_specs=[
            pl.BlockSpec((_HROWS, _W), lambda i: (i, 0)),
            pl.BlockSpec((9, _HROWS, _W), lambda i: (0, i, 0)),
            pl.BlockSpec((1, 10), lambda i: (0, 0)),
            pl.BlockSpec((3, 3), lambda i: (0, 0)),
        ],
        out_specs=pl.BlockSpec((2, 128), lambda i: (0, 0)),
        out_shape=jax.ShapeDtypeStruct((2, 128), jnp.float32),
        scratch_shapes=[
            pltpu.VMEM((4, _NK, _QX, _PX), jnp.float32),
            pltpu.VMEM((4, _NK, _QX, _PX), jnp.float32),
            pltpu.VMEM((_QY, _QX, _PX), jnp.float32),
            pltpu.VMEM((_RPB, _HROWS), jnp.float32),
            pltpu.VMEM((_W, _PX), jnp.float32),
            pltpu.VMEM((_PY, _PX), jnp.float32),
            pltpu.VMEM((2, _PY, _PX), jnp.float32),
            pltpu.VMEM((2, _PY, _PX), jnp.float32),
            pltpu.VMEM((2, _PY, _PX), jnp.float32),
        ],
    )(lab, vert, meta, ext)

    return out[:, :7]


# speedup vs baseline: 1.2405x; 1.2405x over previous
import jax
import jax.numpy as jnp
from jax.experimental import pallas as pl
from jax.experimental.pallas import tpu as pltpu

_SKIP = 5
_H, _W = 480, 640
_PY, _PX = _H // _SKIP, _W // _SKIP
_CSTRIDE = _SKIP * 4
_QY, _QX = _H // _CSTRIDE, _W // _CSTRIDE
_Q = _QY * _QX
_NK = 2 * _QY - 1
_RPB = 8
_NBLK = _PY // _RPB
_HROWS = _RPB * _SKIP

_INLIER_T = 0.5
_ENC2 = 128.0


def _mm(a, b):
    return jax.lax.dot_general(
        a, b, (((1,), (0,)), ((), ())),
        precision=jax.lax.Precision.HIGHEST,
        preferred_element_type=jnp.float32,
    )


def _vote_body(lab_ref, va_ref, vb_ref, meta_ref, ext_ref, out_ref,
               dnx_ref, dny_ref, acc_ref, sr8_ref, sr3_ref, sc_ref,
               labc_ref, uxc_ref, uyc_ref, vzc_ref):
    s = pl.program_id(0)

    @pl.when(s == 0)
    def _init():
        shape = (4, _NK, _QX, _PX)
        b_i = jax.lax.broadcasted_iota(jnp.int32, shape, 0)
        k_i = jax.lax.broadcasted_iota(jnp.int32, shape, 1)
        qx_i = jax.lax.broadcasted_iota(jnp.int32, shape, 2)
        px_i = jax.lax.broadcasted_iota(jnp.int32, shape, 3)
        dyv = (_CSTRIDE * k_i - _CSTRIDE * (_QY - 1) - _SKIP * b_i).astype(
            jnp.float32
        )
        dxv = (_CSTRIDE * qx_i - _SKIP * px_i).astype(jnp.float32)
        dnorm = jnp.sqrt(dxv * dxv + dyv * dyv) + 1e-8
        dnx_ref[...] = dxv / dnorm
        dny_ref[...] = dyv / dnorm
        acc_ref[...] = jnp.zeros_like(acc_ref)

        r_i = jax.lax.broadcasted_iota(jnp.int32, (_RPB, _HROWS), 0)
        x_i = jax.lax.broadcasted_iota(jnp.int32, (_RPB, _HROWS), 1)
        sr8_ref[...] = (x_i == _SKIP * r_i).astype(jnp.float32)
        u_i = jax.lax.broadcasted_iota(jnp.int32, (3 * _RPB, 3 * _HROWS), 0)
        v_i = jax.lax.broadcasted_iota(jnp.int32, (3 * _RPB, 3 * _HROWS), 1)
        sr3_ref[...] = (
            v_i == _HROWS * (u_i // _RPB) + _SKIP * (u_i % _RPB)
        ).astype(jnp.float32)
        w_i = jax.lax.broadcasted_iota(jnp.int32, (_W, _PX), 0)
        j_i = jax.lax.broadcasted_iota(jnp.int32, (_W, _PX), 1)
        sc_ref[...] = (w_i == _SKIP * j_i).astype(jnp.float32)

    @pl.when((s >= 1) & (s <= _NBLK))
    def _vote():
        base = _RPB * (s - 1)
        rows = pl.ds(base, _RPB)
        lab8 = labc_ref[rows, :]
        ux1 = uxc_ref[0, rows, :]
        ux2 = uxc_ref[1, rows, :]
        uy1 = uyc_ref[0, rows, :]
        uy2 = uyc_ref[1, rows, :]
        is1 = lab8 == 1.0
        is2 = lab8 == 2.0
        upx8 = jnp.where(is1, ux1, jnp.where(is2, ux2, 0.0))
        upy8 = jnp.where(is1, uy1, jnp.where(is2, uy2, 0.0))
        encv8 = jnp.where(is1, 1.0, jnp.where(is2, _ENC2, 0.0))

        for r in range(_RPB):
            b = r % 4
            a = 2 * (s - 1) + r // 4
            dnx = dnx_ref[b, pl.ds(_QY - 1 - a, _QY)]
            dny = dny_ref[b, pl.ds(_QY - 1 - a, _QY)]
            upx = upx8[r : r + 1, :].reshape(1, 1, _PX)
            upy = upy8[r : r + 1, :].reshape(1, 1, _PX)
            encv = encv8[r : r + 1, :].reshape(1, 1, _PX)
            dots = dnx * upx + dny * upy
            acc_ref[...] += jnp.where(dots > _INLIER_T, encv, 0.0)

    @pl.when(s < _NBLK)
    def _subsample():
        scm = sc_ref[...]
        va = va_ref[...].reshape(3 * _HROWS, _W)
        vb = vb_ref[...].reshape(3 * _HROWS, _W)
        ya = _mm(sr3_ref[...], _mm(va, scm))
        yb = _mm(sr3_ref[...], _mm(vb, scm))
        lab8 = _mm(sr8_ref[...],
                   _mm(lab_ref[...].astype(jnp.float32), scm))

        vx1 = ya[0:_RPB]
        vy1 = ya[_RPB : 2 * _RPB]
        vz1 = ya[2 * _RPB : 3 * _RPB]
        vx2 = yb[0:_RPB]
        vy2 = yb[_RPB : 2 * _RPB]
        vz2 = yb[2 * _RPB : 3 * _RPB]
        vn1 = jnp.sqrt(vx1 * vx1 + vy1 * vy1) + 1e-8
        vn2 = jnp.sqrt(vx2 * vx2 + vy2 * vy2) + 1e-8

        rows = pl.ds(_RPB * s, _RPB)
        labc_ref[rows, :] = lab8
        uxc_ref[0, rows, :] = vx1 / vn1
        uxc_ref[1, rows, :] = vx2 / vn2
        uyc_ref[0, rows, :] = vy1 / vn1
        uyc_ref[1, rows, :] = vy2 / vn2
        vzc_ref[0, rows, :] = vz1
        vzc_ref[1, rows, :] = vz2

    @pl.when(s == _NBLK)
    def _finish():
        fx = jnp.abs(meta_ref[0:1, 0:1]) + 1.0
        fy = jnp.abs(meta_ref[0:1, 1:2]) + 1.0
        qi = jax.lax.broadcasted_iota(jnp.int32, (_Q, 1), 0)
        lane = jax.lax.broadcasted_iota(jnp.int32, (1, 128), 1)
        z0 = jnp.zeros((1, 128), jnp.float32)

        accf = acc_ref[...].reshape(_Q, _PX)
        v2l = jnp.floor(accf * (1.0 / _ENC2))
        v1l = accf - _ENC2 * v2l

        labf = labc_ref[...]
        pxf = (_SKIP * jax.lax.broadcasted_iota(
            jnp.int32, (_PY, _PX), 1)).astype(jnp.float32)
        pyf = (_SKIP * jax.lax.broadcasted_iota(
            jnp.int32, (_PY, _PX), 0)).astype(jnp.float32)

        for c, vl in ((1, v1l), (2, v2l)):
            votes = jnp.sum(vl, axis=1, keepdims=True)
            m = jnp.max(votes, axis=0, keepdims=True)
            best = jnp.min(
                jnp.where(votes == m, qi, 2**30), axis=0, keepdims=True
            )

            uxf = uxc_ref[c - 1]
            uyf = uyc_ref[c - 1]
            vzf = vzc_ref[c - 1]
            maskf = labf == jnp.float32(c)
            bqx = (_CSTRIDE * (best % _QX)).astype(jnp.float32)
            bqy = (_CSTRIDE * (best // _QX)).astype(jnp.float32)
            dxb = bqx - pxf
            dyb = bqy - pyf
            dnb = jnp.sqrt(dxb * dxb + dyb * dyb) + 1e-8
            dotb = (dxb / dnb) * uxf + (dyb / dnb) * uyf
            hitb = (dotb > _INLIER_T) & maskf
            bvz = jnp.sum(
                jnp.sum(jnp.where(hitb, vzf, 0.0), axis=1, keepdims=True),
                axis=0, keepdims=True,
            )
            cnt = jnp.sum(
                jnp.sum(jnp.where(maskf, 1.0, 0.0), axis=1, keepdims=True),
                axis=0, keepdims=True,
            )

            mean_vz = bvz / (m + 1e-8)
            depth = jnp.exp(jnp.clip(mean_vz, -10.0, 10.0))
            e0 = ext_ref[c : c + 1, 0:1]
            e1 = ext_ref[c : c + 1, 1:2]
            e2 = ext_ref[c : c + 1, 2:3]
            diam = jnp.sqrt(e0 * e0 + e1 * e1 + e2 * e2) + 1e-8
            bw = diam * fx / (depth + 1e-8)
            bh = diam * fy / (depth + 1e-8)
            accept = (
                (m > 0.3 * (cnt + 1e-8)) & (cnt > 5.0)
            ).astype(jnp.float32)
            score = m / (cnt + 1e-8)

            roi = (
                jnp.where(lane == 1, jnp.float32(c), z0)
                + jnp.where(lane == 2, bqx - bw / 2.0, z0)
                + jnp.where(lane == 3, bqy - bh / 2.0, z0)
                + jnp.where(lane == 4, bqx + bw / 2.0, z0)
                + jnp.where(lane == 5, bqy + bh / 2.0, z0)
                + jnp.where(lane == 6, score * accept, z0)
            )
            out_ref[c - 1 : c, :] = roi


@jax.jit
def kernel(bottom_label, bottom_vertex, bottom_meta_data, extents):
    lab = bottom_label[0].astype(jnp.int32)
    vert = bottom_vertex[0].astype(jnp.float32)
    meta = bottom_meta_data.astype(jnp.float32)
    ext = extents.astype(jnp.float32)

    _last = _NBLK - 1
    out = pl.pallas_call(
        _vote_body,
        grid=(_NBLK + 1,),
        in_specs=[
            pl.BlockSpec((_HROWS, _W), lambda i: (jnp.minimum(i, _last), 0)),
            pl.BlockSpec((3, _HROWS, _W),
                         lambda i: (1, jnp.minimum(i, _last), 0)),
            pl.BlockSpec((3, _HROWS, _W),
                         lambda i: (2, jnp.minimum(i, _last), 0)),
            pl.BlockSpec((1, 10), lambda i: (0, 0)),
            pl.BlockSpec((3, 3), lambda i: (0, 0)),
        ],
        out_specs=pl.BlockSpec((2, 128), lambda i: (0, 0)),
        out_shape=jax.ShapeDtypeStruct((2, 128), jnp.float32),
        scratch_shapes=[
            pltpu.VMEM((4, _NK, _QX, _PX), jnp.float32),
            pltpu.VMEM((4, _NK, _QX, _PX), jnp.float32),
            pltpu.VMEM((_QY, _QX, _PX), jnp.float32),
            pltpu.VMEM((_RPB, _HROWS), jnp.float32),
            pltpu.VMEM((3 * _RPB, 3 * _HROWS), jnp.float32),
            pltpu.VMEM((_W, _PX), jnp.float32),
            pltpu.VMEM((_PY, _PX), jnp.float32),
            pltpu.VMEM((2, _PY, _PX), jnp.float32),
            pltpu.VMEM((2, _PY, _PX), jnp.float32),
            pltpu.VMEM((2, _PY, _PX), jnp.float32),
        ],
    )(lab, vert, vert, meta, ext)

    return out[:, :7]


# speedup vs baseline: 1.2569x; 1.0132x over previous
import jax
import jax.numpy as jnp
from jax.experimental import pallas as pl
from jax.experimental.pallas import tpu as pltpu

_SKIP = 5
_H, _W = 480, 640
_PY, _PX = _H // _SKIP, _W // _SKIP
_CSTRIDE = _SKIP * 4
_QY, _QX = _H // _CSTRIDE, _W // _CSTRIDE
_Q = _QY * _QX
_NK = 2 * _QY - 1
_RPB = 8
_NBLK = _PY // _RPB
_HROWS = _RPB * _SKIP

_INLIER_T = 0.5
_ENC2 = 128.0


def _mm(a, b):
    return jax.lax.dot_general(
        a, b, (((1,), (0,)), ((), ())),
        precision=jax.lax.Precision.HIGHEST,
        preferred_element_type=jnp.float32,
    )


def _vote_body(lab_ref, va_ref, vb_ref, meta_ref, ext_ref, out_ref,
               dnx_ref, dny_ref, acc_ref, sr8_ref, sr3_ref, sc_ref,
               labc_ref, uxc_ref, uyc_ref, vzc_ref):
    s = pl.program_id(0)

    @pl.when(s == 0)
    def _init():
        shape = (4, _NK, _QX, _PX)
        b_i = jax.lax.broadcasted_iota(jnp.int32, shape, 0)
        k_i = jax.lax.broadcasted_iota(jnp.int32, shape, 1)
        qx_i = jax.lax.broadcasted_iota(jnp.int32, shape, 2)
        px_i = jax.lax.broadcasted_iota(jnp.int32, shape, 3)
        dyv = (_CSTRIDE * k_i - _CSTRIDE * (_QY - 1) - _SKIP * b_i).astype(
            jnp.float32
        )
        dxv = (_CSTRIDE * qx_i - _SKIP * px_i).astype(jnp.float32)
        dnorm = jnp.sqrt(dxv * dxv + dyv * dyv) + 1e-8
        dnx_ref[...] = dxv / dnorm
        dny_ref[...] = dyv / dnorm
        acc_ref[...] = jnp.zeros_like(acc_ref)

        r_i = jax.lax.broadcasted_iota(jnp.int32, (_RPB, _HROWS), 0)
        x_i = jax.lax.broadcasted_iota(jnp.int32, (_RPB, _HROWS), 1)
        sr8_ref[...] = (x_i == _SKIP * r_i).astype(jnp.float32)
        u_i = jax.lax.broadcasted_iota(jnp.int32, (3 * _RPB, 3 * _HROWS), 0)
        v_i = jax.lax.broadcasted_iota(jnp.int32, (3 * _RPB, 3 * _HROWS), 1)
        sr3_ref[...] = (
            v_i == _HROWS * (u_i // _RPB) + _SKIP * (u_i % _RPB)
        ).astype(jnp.float32)
        w_i = jax.lax.broadcasted_iota(jnp.int32, (_W, _PX), 0)
        j_i = jax.lax.broadcasted_iota(jnp.int32, (_W, _PX), 1)
        sc_ref[...] = (w_i == _SKIP * j_i).astype(jnp.float32)

    sprev = jnp.maximum(s - 1, 0)
    rows_v = pl.ds(_RPB * sprev, _RPB)
    lab8v = labc_ref[rows_v, :]
    ux1v = uxc_ref[0, rows_v, :]
    ux2v = uxc_ref[1, rows_v, :]
    uy1v = uyc_ref[0, rows_v, :]
    uy2v = uyc_ref[1, rows_v, :]
    is1v = lab8v == 1.0
    is2v = lab8v == 2.0
    live = (s >= 1).astype(jnp.float32)
    upx8 = jnp.where(is1v, ux1v, jnp.where(is2v, ux2v, 0.0))
    upy8 = jnp.where(is1v, uy1v, jnp.where(is2v, uy2v, 0.0))
    encv8 = jnp.where(is1v, live, jnp.where(is2v, _ENC2 * live, 0.0))

    for r in range(_RPB):
        b = r % 4
        a = 2 * sprev + r // 4
        dnx = dnx_ref[b, pl.ds(_QY - 1 - a, _QY)]
        dny = dny_ref[b, pl.ds(_QY - 1 - a, _QY)]
        upx = upx8[r : r + 1, :].reshape(1, 1, _PX)
        upy = upy8[r : r + 1, :].reshape(1, 1, _PX)
        encv = encv8[r : r + 1, :].reshape(1, 1, _PX)
        dots = dnx * upx + dny * upy
        acc_ref[...] += jnp.where(dots > _INLIER_T, encv, 0.0)

    scur = jnp.minimum(s, _NBLK - 1)
    scm = sc_ref[...]
    va = va_ref[...].reshape(3 * _HROWS, _W)
    vb = vb_ref[...].reshape(3 * _HROWS, _W)
    ya = _mm(_mm(sr3_ref[...], va), scm)
    yb = _mm(_mm(sr3_ref[...], vb), scm)
    lab8 = _mm(_mm(sr8_ref[...],
                   lab_ref[...].astype(jnp.float32)), scm)

    vx1 = ya[0:_RPB]
    vy1 = ya[_RPB : 2 * _RPB]
    vz1 = ya[2 * _RPB : 3 * _RPB]
    vx2 = yb[0:_RPB]
    vy2 = yb[_RPB : 2 * _RPB]
    vz2 = yb[2 * _RPB : 3 * _RPB]
    vn1 = jnp.sqrt(vx1 * vx1 + vy1 * vy1) + 1e-8
    vn2 = jnp.sqrt(vx2 * vx2 + vy2 * vy2) + 1e-8

    rows = pl.ds(_RPB * scur, _RPB)
    labc_ref[rows, :] = lab8
    uxc_ref[0, rows, :] = vx1 / vn1
    uxc_ref[1, rows, :] = vx2 / vn2
    uyc_ref[0, rows, :] = vy1 / vn1
    uyc_ref[1, rows, :] = vy2 / vn2
    vzc_ref[0, rows, :] = vz1
    vzc_ref[1, rows, :] = vz2

    @pl.when(s == _NBLK)
    def _finish():
        fx = jnp.abs(meta_ref[0:1, 0:1]) + 1.0
        fy = jnp.abs(meta_ref[0:1, 1:2]) + 1.0
        qi = jax.lax.broadcasted_iota(jnp.int32, (_Q, 1), 0)
        lane = jax.lax.broadcasted_iota(jnp.int32, (1, 128), 1)
        z0 = jnp.zeros((1, 128), jnp.float32)

        accf = acc_ref[...].reshape(_Q, _PX)
        v2l = jnp.floor(accf * (1.0 / _ENC2))
        v1l = accf - _ENC2 * v2l

        labf = labc_ref[...]
        pxf = (_SKIP * jax.lax.broadcasted_iota(
            jnp.int32, (_PY, _PX), 1)).astype(jnp.float32)
        pyf = (_SKIP * jax.lax.broadcasted_iota(
            jnp.int32, (_PY, _PX), 0)).astype(jnp.float32)

        for c, vl in ((1, v1l), (2, v2l)):
            votes = jnp.sum(vl, axis=1, keepdims=True)
            m = jnp.max(votes, axis=0, keepdims=True)
            best = jnp.min(
                jnp.where(votes == m, qi, 2**30), axis=0, keepdims=True
            )

            uxf = uxc_ref[c - 1]
            uyf = uyc_ref[c - 1]
            vzf = vzc_ref[c - 1]
            maskf = labf == jnp.float32(c)
            bqx = (_CSTRIDE * (best % _QX)).astype(jnp.float32)
            bqy = (_CSTRIDE * (best // _QX)).astype(jnp.float32)
            dxb = bqx - pxf
            dyb = bqy - pyf
            dnb = jnp.sqrt(dxb * dxb + dyb * dyb) + 1e-8
            dotb = (dxb / dnb) * uxf + (dyb / dnb) * uyf
            hitb = (dotb > _INLIER_T) & maskf
            bvz = jnp.sum(
                jnp.sum(jnp.where(hitb, vzf, 0.0), axis=1, keepdims=True),
                axis=0, keepdims=True,
            )
            cnt = jnp.sum(
                jnp.sum(jnp.where(maskf, 1.0, 0.0), axis=1, keepdims=True),
                axis=0, keepdims=True,
            )

            mean_vz = bvz / (m + 1e-8)
            depth = jnp.exp(jnp.clip(mean_vz, -10.0, 10.0))
            e0 = ext_ref[c : c + 1, 0:1]
            e1 = ext_ref[c : c + 1, 1:2]
            e2 = ext_ref[c : c + 1, 2:3]
            diam = jnp.sqrt(e0 * e0 + e1 * e1 + e2 * e2) + 1e-8
            bw = diam * fx / (depth + 1e-8)
            bh = diam * fy / (depth + 1e-8)
            accept = (
                (m > 0.3 * (cnt + 1e-8)) & (cnt > 5.0)
            ).astype(jnp.float32)
            score = m / (cnt + 1e-8)

            roi = (
                jnp.where(lane == 1, jnp.float32(c), z0)
                + jnp.where(lane == 2, bqx - bw / 2.0, z0)
                + jnp.where(lane == 3, bqy - bh / 2.0, z0)
                + jnp.where(lane == 4, bqx + bw / 2.0, z0)
                + jnp.where(lane == 5, bqy + bh / 2.0, z0)
                + jnp.where(lane == 6, score * accept, z0)
            )
            out_ref[c - 1 : c, :] = roi


@jax.jit
def kernel(bottom_label, bottom_vertex, bottom_meta_data, extents):
    lab = bottom_label[0].astype(jnp.int32)
    vert = bottom_vertex[0].astype(jnp.float32)
    meta = bottom_meta_data.astype(jnp.float32)
    ext = extents.astype(jnp.float32)

    _last = _NBLK - 1
    out = pl.pallas_call(
        _vote_body,
        grid=(_NBLK + 1,),
        in_specs=[
            pl.BlockSpec((_HROWS, _W), lambda i: (jnp.minimum(i, _last), 0)),
            pl.BlockSpec((3, _HROWS, _W),
                         lambda i: (1, jnp.minimum(i, _last), 0)),
            pl.BlockSpec((3, _HROWS, _W),
                         lambda i: (2, jnp.minimum(i, _last), 0)),
            pl.BlockSpec((1, 10), lambda i: (0, 0)),
            pl.BlockSpec((3, 3), lambda i: (0, 0)),
        ],
        out_specs=pl.BlockSpec((2, 128), lambda i: (0, 0)),
        out_shape=jax.ShapeDtypeStruct((2, 128), jnp.float32),
        scratch_shapes=[
            pltpu.VMEM((4, _NK, _QX, _PX), jnp.float32),
            pltpu.VMEM((4, _NK, _QX, _PX), jnp.float32),
            pltpu.VMEM((_QY, _QX, _PX), jnp.float32),
            pltpu.VMEM((_RPB, _HROWS), jnp.float32),
            pltpu.VMEM((3 * _RPB, 3 * _HROWS), jnp.float32),
            pltpu.VMEM((_W, _PX), jnp.float32),
            pltpu.VMEM((_PY, _PX), jnp.float32),
            pltpu.VMEM((2, _PY, _PX), jnp.float32),
            pltpu.VMEM((2, _PY, _PX), jnp.float32),
            pltpu.VMEM((2, _PY, _PX), jnp.float32),
        ],
    )(lab, vert, vert, meta, ext)

    return out[:, :7]


# speedup vs baseline: 1.3272x; 1.0560x over previous
import jax
import jax.numpy as jnp
from jax.experimental import pallas as pl
from jax.experimental.pallas import tpu as pltpu

_SKIP = 5
_H, _W = 480, 640
_PY, _PX = _H // _SKIP, _W // _SKIP
_CSTRIDE = _SKIP * 4
_QY, _QX = _H // _CSTRIDE, _W // _CSTRIDE
_Q = _QY * _QX
_NK = 2 * _QY - 1
_RPB = 16
_NBLK = _PY // _RPB
_HROWS = _RPB * _SKIP

_INLIER_T = 0.5
_ENC2 = 128.0


def _mm(a, b):
    return jax.lax.dot_general(
        a, b, (((1,), (0,)), ((), ())),
        precision=jax.lax.Precision.HIGHEST,
        preferred_element_type=jnp.float32,
    )


def _vote_body(lab_ref, va_ref, vb_ref, meta_ref, ext_ref, out_ref,
               dnx_ref, dny_ref, acc_ref, sr8_ref, sr3_ref, sc_ref,
               labc_ref, uxc_ref, uyc_ref, vzc_ref):
    s = pl.program_id(0)

    @pl.when(s == 0)
    def _init():
        shape = (4, _NK, _QX, _PX)
        b_i = jax.lax.broadcasted_iota(jnp.int32, shape, 0)
        k_i = jax.lax.broadcasted_iota(jnp.int32, shape, 1)
        qx_i = jax.lax.broadcasted_iota(jnp.int32, shape, 2)
        px_i = jax.lax.broadcasted_iota(jnp.int32, shape, 3)
        dyv = (_CSTRIDE * k_i - _CSTRIDE * (_QY - 1) - _SKIP * b_i).astype(
            jnp.float32
        )
        dxv = (_CSTRIDE * qx_i - _SKIP * px_i).astype(jnp.float32)
        dnorm = jnp.sqrt(dxv * dxv + dyv * dyv) + 1e-8
        dnx_ref[...] = dxv / dnorm
        dny_ref[...] = dyv / dnorm
        acc_ref[...] = jnp.zeros_like(acc_ref)

        r_i = jax.lax.broadcasted_iota(jnp.int32, (_RPB, _HROWS), 0)
        x_i = jax.lax.broadcasted_iota(jnp.int32, (_RPB, _HROWS), 1)
        sr8_ref[...] = (x_i == _SKIP * r_i).astype(jnp.float32)
        u_i = jax.lax.broadcasted_iota(jnp.int32, (3 * _RPB, 3 * _HROWS), 0)
        v_i = jax.lax.broadcasted_iota(jnp.int32, (3 * _RPB, 3 * _HROWS), 1)
        sr3_ref[...] = (
            v_i == _HROWS * (u_i // _RPB) + _SKIP * (u_i % _RPB)
        ).astype(jnp.float32)
        w_i = jax.lax.broadcasted_iota(jnp.int32, (_W, _PX), 0)
        j_i = jax.lax.broadcasted_iota(jnp.int32, (_W, _PX), 1)
        sc_ref[...] = (w_i == _SKIP * j_i).astype(jnp.float32)

    sprev = jnp.maximum(s - 1, 0)
    rows_v = pl.ds(_RPB * sprev, _RPB)
    lab8v = labc_ref[rows_v, :]
    ux1v = uxc_ref[0, rows_v, :]
    ux2v = uxc_ref[1, rows_v, :]
    uy1v = uyc_ref[0, rows_v, :]
    uy2v = uyc_ref[1, rows_v, :]
    is1v = lab8v == 1.0
    is2v = lab8v == 2.0
    live = (s >= 1).astype(jnp.float32)
    upx8 = jnp.where(is1v, ux1v, jnp.where(is2v, ux2v, 0.0))
    upy8 = jnp.where(is1v, uy1v, jnp.where(is2v, uy2v, 0.0))
    encv8 = jnp.where(is1v, live, jnp.where(is2v, _ENC2 * live, 0.0))

    for r in range(_RPB):
        b = r % 4
        a = (_RPB // 4) * sprev + r // 4
        dnx = dnx_ref[b, pl.ds(_QY - 1 - a, _QY)]
        dny = dny_ref[b, pl.ds(_QY - 1 - a, _QY)]
        upx = upx8[r : r + 1, :].reshape(1, 1, _PX)
        upy = upy8[r : r + 1, :].reshape(1, 1, _PX)
        encv = encv8[r : r + 1, :].reshape(1, 1, _PX)
        dots = dnx * upx + dny * upy
        acc_ref[...] += jnp.where(dots > _INLIER_T, encv, 0.0)

    scur = jnp.minimum(s, _NBLK - 1)
    scm = sc_ref[...]
    va = va_ref[...].reshape(3 * _HROWS, _W)
    vb = vb_ref[...].reshape(3 * _HROWS, _W)
    ya = _mm(_mm(sr3_ref[...], va), scm)
    yb = _mm(_mm(sr3_ref[...], vb), scm)
    lab8 = _mm(_mm(sr8_ref[...],
                   lab_ref[...].astype(jnp.float32)), scm)

    vx1 = ya[0:_RPB]
    vy1 = ya[_RPB : 2 * _RPB]
    vz1 = ya[2 * _RPB : 3 * _RPB]
    vx2 = yb[0:_RPB]
    vy2 = yb[_RPB : 2 * _RPB]
    vz2 = yb[2 * _RPB : 3 * _RPB]
    vn1 = jnp.sqrt(vx1 * vx1 + vy1 * vy1) + 1e-8
    vn2 = jnp.sqrt(vx2 * vx2 + vy2 * vy2) + 1e-8

    rows = pl.ds(_RPB * scur, _RPB)
    labc_ref[rows, :] = lab8
    uxc_ref[0, rows, :] = vx1 / vn1
    uxc_ref[1, rows, :] = vx2 / vn2
    uyc_ref[0, rows, :] = vy1 / vn1
    uyc_ref[1, rows, :] = vy2 / vn2
    vzc_ref[0, rows, :] = vz1
    vzc_ref[1, rows, :] = vz2

    @pl.when(s == _NBLK)
    def _finish():
        fx = jnp.abs(meta_ref[0:1, 0:1]) + 1.0
        fy = jnp.abs(meta_ref[0:1, 1:2]) + 1.0
        qi = jax.lax.broadcasted_iota(jnp.int32, (_Q, 1), 0)
        lane = jax.lax.broadcasted_iota(jnp.int32, (1, 128), 1)
        z0 = jnp.zeros((1, 128), jnp.float32)

        accf = acc_ref[...].reshape(_Q, _PX)
        v2l = jnp.floor(accf * (1.0 / _ENC2))
        v1l = accf - _ENC2 * v2l

        labf = labc_ref[...]
        pxf = (_SKIP * jax.lax.broadcasted_iota(
            jnp.int32, (_PY, _PX), 1)).astype(jnp.float32)
        pyf = (_SKIP * jax.lax.broadcasted_iota(
            jnp.int32, (_PY, _PX), 0)).astype(jnp.float32)

        for c, vl in ((1, v1l), (2, v2l)):
            votes = jnp.sum(vl, axis=1, keepdims=True)
            m = jnp.max(votes, axis=0, keepdims=True)
            best = jnp.min(
                jnp.where(votes == m, qi, 2**30), axis=0, keepdims=True
            )

            uxf = uxc_ref[c - 1]
            uyf = uyc_ref[c - 1]
            vzf = vzc_ref[c - 1]
            maskf = labf == jnp.float32(c)
            bqx = (_CSTRIDE * (best % _QX)).astype(jnp.float32)
            bqy = (_CSTRIDE * (best // _QX)).astype(jnp.float32)
            dxb = bqx - pxf
            dyb = bqy - pyf
            dnb = jnp.sqrt(dxb * dxb + dyb * dyb) + 1e-8
            dotb = (dxb / dnb) * uxf + (dyb / dnb) * uyf
            hitb = (dotb > _INLIER_T) & maskf
            bvz = jnp.sum(
                jnp.sum(jnp.where(hitb, vzf, 0.0), axis=1, keepdims=True),
                axis=0, keepdims=True,
            )
            cnt = jnp.sum(
                jnp.sum(jnp.where(maskf, 1.0, 0.0), axis=1, keepdims=True),
                axis=0, keepdims=True,
            )

            mean_vz = bvz / (m + 1e-8)
            depth = jnp.exp(jnp.clip(mean_vz, -10.0, 10.0))
            e0 = ext_ref[c : c + 1, 0:1]
            e1 = ext_ref[c : c + 1, 1:2]
            e2 = ext_ref[c : c + 1, 2:3]
            diam = jnp.sqrt(e0 * e0 + e1 * e1 + e2 * e2) + 1e-8
            bw = diam * fx / (depth + 1e-8)
            bh = diam * fy / (depth + 1e-8)
            accept = (
                (m > 0.3 * (cnt + 1e-8)) & (cnt > 5.0)
            ).astype(jnp.float32)
            score = m / (cnt + 1e-8)

            roi = (
                jnp.where(lane == 1, jnp.float32(c), z0)
                + jnp.where(lane == 2, bqx - bw / 2.0, z0)
                + jnp.where(lane == 3, bqy - bh / 2.0, z0)
                + jnp.where(lane == 4, bqx + bw / 2.0, z0)
                + jnp.where(lane == 5, bqy + bh / 2.0, z0)
                + jnp.where(lane == 6, score * accept, z0)
            )
            out_ref[c - 1 : c, :] = roi


@jax.jit
def kernel(bottom_label, bottom_vertex, bottom_meta_data, extents):
    lab = bottom_label[0].astype(jnp.int32)
    vert = bottom_vertex[0].astype(jnp.float32)
    meta = bottom_meta_data.astype(jnp.float32)
    ext = extents.astype(jnp.float32)

    _last = _NBLK - 1
    out = pl.pallas_call(
        _vote_body,
        grid=(_NBLK + 1,),
        in_specs=[
            pl.BlockSpec((_HROWS, _W), lambda i: (jnp.minimum(i, _last), 0)),
            pl.BlockSpec((3, _HROWS, _W),
                         lambda i: (1, jnp.minimum(i, _last), 0)),
            pl.BlockSpec((3, _HROWS, _W),
                         lambda i: (2, jnp.minimum(i, _last), 0)),
            pl.BlockSpec((1, 10), lambda i: (0, 0)),
            pl.BlockSpec((3, 3), lambda i: (0, 0)),
        ],
        out_specs=pl.BlockSpec((2, 128), lambda i: (0, 0)),
        out_shape=jax.ShapeDtypeStruct((2, 128), jnp.float32),
        scratch_shapes=[
            pltpu.VMEM((4, _NK, _QX, _PX), jnp.float32),
            pltpu.VMEM((4, _NK, _QX, _PX), jnp.float32),
            pltpu.VMEM((_QY, _QX, _PX), jnp.float32),
            pltpu.VMEM((_RPB, _HROWS), jnp.float32),
            pltpu.VMEM((3 * _RPB, 3 * _HROWS), jnp.float32),
            pltpu.VMEM((_W, _PX), jnp.float32),
            pltpu.VMEM((_PY, _PX), jnp.float32),
            pltpu.VMEM((2, _PY, _PX), jnp.float32),
            pltpu.VMEM((2, _PY, _PX), jnp.float32),
            pltpu.VMEM((2, _PY, _PX), jnp.float32),
        ],
    )(lab, vert, vert, meta, ext)

    return out[:, :7]
